# baseline (speedup 1.0000x reference)
import functools

import jax
import jax.numpy as jnp
from jax import lax
from jax.experimental import pallas as pl
from jax.experimental.pallas import tpu as pltpu
from jax.experimental.pallas import tpu_sc as plsc

_N = 10000
_D = 128
_E = 320000

_CH = 128
_NW = 32
_CA = 91
_CB = 67
_CHUNKS = max(_CA, _CB)
_EPAD = (_CA + _CB) * 16 * _CH
_NROWS = 10240
_RPT = _NROWS // 16
_OPT = _N // 16

_BLK = 1000
_GRID = _N // _BLK


def _sc_body(rx_hbm, row_hbm, col_hbm, out_hbm, row_v, col_v, rows_v, sem,
             shared):
    c = lax.axis_index("c")
    s = lax.axis_index("s")
    wid = c * 16 + s

    pltpu.sync_copy(row_hbm.at[wid], row_v)
    pltpu.sync_copy(col_hbm.at[wid], col_v)

    def _zr(r, carry):
        for c8 in range(_D // 16):
            rows_v[r, pl.ds(c8 * 16, 16)] = jnp.zeros((16,), jnp.float32)
        return carry

    lax.fori_loop(0, _CH, _zr, 0)
    for j in range(_RPT // _CH):
        pltpu.sync_copy(rows_v, shared.at[pl.ds(s * _RPT + j * _CH, _CH)])
    plsc.subcore_barrier()

    def _ebody(j, carry):
        pltpu.sync_copy(rx_hbm.at[row_v.at[j]], rows_v)
        pltpu.sync_copy(rows_v, shared.at[col_v.at[j]], add=True)
        return carry

    trip = jnp.where(c == 0, _CA, _CB)
    lax.fori_loop(0, trip, _ebody, 0)
    plsc.subcore_barrier()

    pltpu.sync_copy(shared.at[pl.ds(s * _RPT, _RPT)],
                    out_hbm.at[c, pl.ds(s * _RPT, _RPT)])


@functools.cache
def _sc_agg():
    return pl.kernel(
        _sc_body,
        out_type=jax.ShapeDtypeStruct((2, _NROWS, _D), jnp.float32),
        mesh=plsc.VectorSubcoreMesh(core_axis_name="c", subcore_axis_name="s"),
        scratch_types=[
            pltpu.VMEM((_CHUNKS, _CH), jnp.int32),
            pltpu.VMEM((_CHUNKS, _CH), jnp.int32),
            pltpu.VMEM((_CH, _D), jnp.float32),
            pltpu.SemaphoreType.DMA,
            pltpu.VMEM_SHARED((_NROWS, _D), jnp.float32),
        ],
    )


def _relu_body(x_ref, o_ref):
    o_ref[...] = jnp.maximum(x_ref[...], 0.0)


def _mlp1_body(eps_ref, x_ref, agg_ref, w1_ref, b1_ref, h1_ref, st_ref):
    i = pl.program_id(0)
    h = (1.0 + eps_ref[0, 0]) * x_ref[...] + agg_ref[0] + agg_ref[1]
    h1 = lax.dot_general(h, w1_ref[...], (((1,), (1,)), ((), ())),
                         preferred_element_type=jnp.float32) + b1_ref[...]
    h1_ref[...] = h1
    st = jnp.concatenate(
        [jnp.sum(h1, axis=0)[None], jnp.sum(h1 * h1, axis=0)[None]], axis=0)

    @pl.when(i == 0)
    def _init():
        st_ref[...] = st

    @pl.when(i > 0)
    def _acc():
        st_ref[...] += st


def _mlp2_body(st_ref, gamma_ref, beta_ref, h1_ref, w2_ref, b2_ref, o_ref):
    mean = st_ref[0, :] / _N
    var = st_ref[1, :] / _N - mean * mean
    scale = gamma_ref[0] * lax.rsqrt(var + 1e-5)
    shift = beta_ref[0] - mean * scale
    h2 = jnp.maximum(h1_ref[...] * scale + shift, 0.0)
    o_ref[...] = lax.dot_general(h2, w2_ref[...], (((1,), (1,)), ((), ())),
                                 preferred_element_type=jnp.float32) + b2_ref[...]


def kernel(x, edge_index, W1, b1, gamma, beta, W2, b2, eps):
    row = edge_index[0]
    col = edge_index[1]
    pad = _EPAD - _E
    trash = _N + (jnp.arange(pad, dtype=jnp.int32) % (_NROWS - _N))
    e0 = 16 * _CA * _CH

    def _split(flat):
        p0 = flat[:e0].reshape(16, _CA, _CH)
        p1 = flat[e0:].reshape(16, _CB, _CH)
        if _CA < _CHUNKS:
            p0 = jnp.concatenate(
                [p0, jnp.zeros((16, _CHUNKS - _CA, _CH), jnp.int32)], axis=1)
        if _CB < _CHUNKS:
            p1 = jnp.concatenate(
                [p1, jnp.zeros((16, _CHUNKS - _CB, _CH), jnp.int32)], axis=1)
        return jnp.concatenate([p0, p1], axis=0)

    row_p = _split(jnp.concatenate([row, jnp.zeros((pad,), jnp.int32)]))
    col_p = _split(jnp.concatenate([col, trash]))

    rx = pl.pallas_call(
        _relu_body,
        grid=(_GRID,),
        in_specs=[pl.BlockSpec((_BLK, _D), lambda i: (i, 0))],
        out_specs=pl.BlockSpec((_BLK, _D), lambda i: (i, 0)),
        out_shape=jax.ShapeDtypeStruct((_N, _D), jnp.float32),
    )(x)

    agg2 = _sc_agg()(rx, row_p, col_p)

    h1, st = pl.pallas_call(
        _mlp1_body,
        grid=(_GRID,),
        in_specs=[
            pl.BlockSpec(memory_space=pltpu.SMEM),
            pl.BlockSpec((_BLK, _D), lambda i: (i, 0)),
            pl.BlockSpec((2, _BLK, _D), lambda i: (0, i, 0)),
            pl.BlockSpec((2 * _D, _D), lambda i: (0, 0)),
            pl.BlockSpec((1, 2 * _D), lambda i: (0, 0)),
        ],
        out_specs=[
            pl.BlockSpec((_BLK, 2 * _D), lambda i: (i, 0)),
            pl.BlockSpec((2, 2 * _D), lambda i: (0, 0)),
        ],
        out_shape=[
            jax.ShapeDtypeStruct((_N, 2 * _D), jnp.float32),
            jax.ShapeDtypeStruct((2, 2 * _D), jnp.float32),
        ],
    )(eps.reshape(1, 1), x, agg2, W1, b1.reshape(1, 2 * _D))

    out = pl.pallas_call(
        _mlp2_body,
        grid=(_GRID,),
        in_specs=[
            pl.BlockSpec((2, 2 * _D), lambda i: (0, 0)),
            pl.BlockSpec((1, 2 * _D), lambda i: (0, 0)),
            pl.BlockSpec((1, 2 * _D), lambda i: (0, 0)),
            pl.BlockSpec((_BLK, 2 * _D), lambda i: (i, 0)),
            pl.BlockSpec((_D, 2 * _D), lambda i: (0, 0)),
            pl.BlockSpec((1, _D), lambda i: (0, 0)),
        ],
        out_specs=pl.BlockSpec((_BLK, _D), lambda i: (i, 0)),
        out_shape=jax.ShapeDtypeStruct((_N, _D), jnp.float32),
    )(st, gamma.reshape(1, 2 * _D), beta.reshape(1, 2 * _D), h1, W2,
      b2.reshape(1, _D))

    return out


# speedup vs baseline: 1.0818x; 1.0818x over previous
import functools

import jax
import jax.numpy as jnp
from jax import lax
from jax.experimental import pallas as pl
from jax.experimental.pallas import tpu as pltpu
from jax.experimental.pallas import tpu_sc as plsc

_N = 10000
_D = 128
_E = 320000

_CH = 128
_NW = 32
_CA = 115
_CB = 43
_CHUNKS = max(_CA, _CB)
_EPAD = (_CA + _CB) * 16 * _CH
_NROWS = 10240
_RPT = _NROWS // 16
_OPT = _N // 16

_BLK = 1000
_GRID = _N // _BLK


def _sc_body(rx_hbm, row_hbm, col_hbm, out_hbm, row_v, col_v, rows_v, sem,
             shared):
    c = lax.axis_index("c")
    s = lax.axis_index("s")
    wid = c * 16 + s

    pltpu.sync_copy(row_hbm.at[wid], row_v)
    pltpu.sync_copy(col_hbm.at[wid], col_v)

    def _zr(r, carry):
        for c8 in range(_D // 16):
            rows_v[r, pl.ds(c8 * 16, 16)] = jnp.zeros((16,), jnp.float32)
        return carry

    lax.fori_loop(0, _CH, _zr, 0)
    for j in range(_RPT // _CH):
        pltpu.sync_copy(rows_v, shared.at[pl.ds(s * _RPT + j * _CH, _CH)])
    plsc.subcore_barrier()

    def _ebody(j, carry):
        pltpu.sync_copy(rx_hbm.at[row_v.at[j]], rows_v)
        pltpu.sync_copy(rows_v, shared.at[col_v.at[j]], add=True)
        return carry

    trip = jnp.where(c == 0, _CA, _CB)
    lax.fori_loop(0, trip, _ebody, 0)
    plsc.subcore_barrier()

    pltpu.sync_copy(shared.at[pl.ds(s * _RPT, _RPT)],
                    out_hbm.at[c, pl.ds(s * _RPT, _RPT)])


@functools.cache
def _sc_agg():
    return pl.kernel(
        _sc_body,
        out_type=jax.ShapeDtypeStruct((2, _NROWS, _D), jnp.float32),
        mesh=plsc.VectorSubcoreMesh(core_axis_name="c", subcore_axis_name="s"),
        scratch_types=[
            pltpu.VMEM((_CHUNKS, _CH), jnp.int32),
            pltpu.VMEM((_CHUNKS, _CH), jnp.int32),
            pltpu.VMEM((_CH, _D), jnp.float32),
            pltpu.SemaphoreType.DMA,
            pltpu.VMEM_SHARED((_NROWS, _D), jnp.float32),
        ],
    )


def _relu_body(x_ref, o_ref):
    o_ref[...] = jnp.maximum(x_ref[...], 0.0)


def _mlp1_body(eps_ref, x_ref, agg_ref, w1_ref, b1_ref, h1_ref, st_ref):
    i = pl.program_id(0)
    h = (1.0 + eps_ref[0, 0]) * x_ref[...] + agg_ref[0] + agg_ref[1]
    h1 = lax.dot_general(h, w1_ref[...], (((1,), (1,)), ((), ())),
                         preferred_element_type=jnp.float32) + b1_ref[...]
    h1_ref[...] = h1
    st = jnp.concatenate(
        [jnp.sum(h1, axis=0)[None], jnp.sum(h1 * h1, axis=0)[None]], axis=0)

    @pl.when(i == 0)
    def _init():
        st_ref[...] = st

    @pl.when(i > 0)
    def _acc():
        st_ref[...] += st


def _mlp2_body(st_ref, gamma_ref, beta_ref, h1_ref, w2_ref, b2_ref, o_ref):
    mean = st_ref[0, :] / _N
    var = st_ref[1, :] / _N - mean * mean
    scale = gamma_ref[0] * lax.rsqrt(var + 1e-5)
    shift = beta_ref[0] - mean * scale
    h2 = jnp.maximum(h1_ref[...] * scale + shift, 0.0)
    o_ref[...] = lax.dot_general(h2, w2_ref[...], (((1,), (1,)), ((), ())),
                                 preferred_element_type=jnp.float32) + b2_ref[...]


def kernel(x, edge_index, W1, b1, gamma, beta, W2, b2, eps):
    row = edge_index[0]
    col = edge_index[1]
    pad = _EPAD - _E
    trash = _N + (jnp.arange(pad, dtype=jnp.int32) % (_NROWS - _N))
    e0 = 16 * _CA * _CH

    def _split(flat):
        p0 = flat[:e0].reshape(16, _CA, _CH)
        p1 = flat[e0:].reshape(16, _CB, _CH)
        if _CA < _CHUNKS:
            p0 = jnp.concatenate(
                [p0, jnp.zeros((16, _CHUNKS - _CA, _CH), jnp.int32)], axis=1)
        if _CB < _CHUNKS:
            p1 = jnp.concatenate(
                [p1, jnp.zeros((16, _CHUNKS - _CB, _CH), jnp.int32)], axis=1)
        return jnp.concatenate([p0, p1], axis=0)

    row_p = _split(jnp.concatenate([row, jnp.zeros((pad,), jnp.int32)]))
    col_p = _split(jnp.concatenate([col, trash]))

    rx = pl.pallas_call(
        _relu_body,
        grid=(_GRID,),
        in_specs=[pl.BlockSpec((_BLK, _D), lambda i: (i, 0))],
        out_specs=pl.BlockSpec((_BLK, _D), lambda i: (i, 0)),
        out_shape=jax.ShapeDtypeStruct((_N, _D), jnp.float32),
    )(x)

    agg2 = _sc_agg()(rx, row_p, col_p)

    h1, st = pl.pallas_call(
        _mlp1_body,
        grid=(_GRID,),
        in_specs=[
            pl.BlockSpec(memory_space=pltpu.SMEM),
            pl.BlockSpec((_BLK, _D), lambda i: (i, 0)),
            pl.BlockSpec((2, _BLK, _D), lambda i: (0, i, 0)),
            pl.BlockSpec((2 * _D, _D), lambda i: (0, 0)),
            pl.BlockSpec((1, 2 * _D), lambda i: (0, 0)),
        ],
        out_specs=[
            pl.BlockSpec((_BLK, 2 * _D), lambda i: (i, 0)),
            pl.BlockSpec((2, 2 * _D), lambda i: (0, 0)),
        ],
        out_shape=[
            jax.ShapeDtypeStruct((_N, 2 * _D), jnp.float32),
            jax.ShapeDtypeStruct((2, 2 * _D), jnp.float32),
        ],
    )(eps.reshape(1, 1), x, agg2, W1, b1.reshape(1, 2 * _D))

    out = pl.pallas_call(
        _mlp2_body,
        grid=(_GRID,),
        in_specs=[
            pl.BlockSpec((2, 2 * _D), lambda i: (0, 0)),
            pl.BlockSpec((1, 2 * _D), lambda i: (0, 0)),
            pl.BlockSpec((1, 2 * _D), lambda i: (0, 0)),
            pl.BlockSpec((_BLK, 2 * _D), lambda i: (i, 0)),
            pl.BlockSpec((_D, 2 * _D), lambda i: (0, 0)),
            pl.BlockSpec((1, _D), lambda i: (0, 0)),
        ],
        out_specs=pl.BlockSpec((_BLK, _D), lambda i: (i, 0)),
        out_shape=jax.ShapeDtypeStruct((_N, _D), jnp.float32),
    )(st, gamma.reshape(1, 2 * _D), beta.reshape(1, 2 * _D), h1, W2,
      b2.reshape(1, _D))

    return out


# speedup vs baseline: 1.0859x; 1.0038x over previous
import functools

import jax
import jax.numpy as jnp
from jax import lax
from jax.experimental import pallas as pl
from jax.experimental.pallas import tpu as pltpu
from jax.experimental.pallas import tpu_sc as plsc

_N = 10000
_D = 128
_E = 320000

_CH = 128
_NW = 32
_CA = 128
_CB = 30
_CHUNKS = max(_CA, _CB)
_EPAD = (_CA + _CB) * 16 * _CH
_NROWS = 10240
_RPT = _NROWS // 16
_OPT = _N // 16

_BLK = 1000
_GRID = _N // _BLK


def _sc_body(rx_hbm, row_hbm, col_hbm, out_hbm, row_v, col_v, rows_v, sem,
             shared):
    c = lax.axis_index("c")
    s = lax.axis_index("s")
    wid = c * 16 + s

    pltpu.sync_copy(row_hbm.at[wid], row_v)
    pltpu.sync_copy(col_hbm.at[wid], col_v)

    def _zr(r, carry):
        for c8 in range(_D // 16):
            rows_v[r, pl.ds(c8 * 16, 16)] = jnp.zeros((16,), jnp.float32)
        return carry

    lax.fori_loop(0, _CH, _zr, 0)
    for j in range(_RPT // _CH):
        pltpu.sync_copy(rows_v, shared.at[pl.ds(s * _RPT + j * _CH, _CH)])
    plsc.subcore_barrier()

    def _ebody(j, carry):
        pltpu.sync_copy(rx_hbm.at[row_v.at[j]], rows_v)
        pltpu.sync_copy(rows_v, shared.at[col_v.at[j]], add=True)
        return carry

    trip = jnp.where(c == 0, _CA, _CB)
    lax.fori_loop(0, trip, _ebody, 0)
    plsc.subcore_barrier()

    pltpu.sync_copy(shared.at[pl.ds(s * _RPT, _RPT)],
                    out_hbm.at[c, pl.ds(s * _RPT, _RPT)])


@functools.cache
def _sc_agg():
    return pl.kernel(
        _sc_body,
        out_type=jax.ShapeDtypeStruct((2, _NROWS, _D), jnp.float32),
        mesh=plsc.VectorSubcoreMesh(core_axis_name="c", subcore_axis_name="s"),
        scratch_types=[
            pltpu.VMEM((_CHUNKS, _CH), jnp.int32),
            pltpu.VMEM((_CHUNKS, _CH), jnp.int32),
            pltpu.VMEM((_CH, _D), jnp.float32),
            pltpu.SemaphoreType.DMA,
            pltpu.VMEM_SHARED((_NROWS, _D), jnp.float32),
        ],
    )


def _relu_body(x_ref, o_ref):
    o_ref[...] = jnp.maximum(x_ref[...], 0.0)


def _mlp1_body(eps_ref, x_ref, agg_ref, w1_ref, b1_ref, h1_ref, st_ref):
    i = pl.program_id(0)
    h = (1.0 + eps_ref[0, 0]) * x_ref[...] + agg_ref[0] + agg_ref[1]
    h1 = lax.dot_general(h, w1_ref[...], (((1,), (1,)), ((), ())),
                         preferred_element_type=jnp.float32) + b1_ref[...]
    h1_ref[...] = h1
    st = jnp.concatenate(
        [jnp.sum(h1, axis=0)[None], jnp.sum(h1 * h1, axis=0)[None]], axis=0)

    @pl.when(i == 0)
    def _init():
        st_ref[...] = st

    @pl.when(i > 0)
    def _acc():
        st_ref[...] += st


def _mlp2_body(st_ref, gamma_ref, beta_ref, h1_ref, w2_ref, b2_ref, o_ref):
    mean = st_ref[0, :] / _N
    var = st_ref[1, :] / _N - mean * mean
    scale = gamma_ref[0] * lax.rsqrt(var + 1e-5)
    shift = beta_ref[0] - mean * scale
    h2 = jnp.maximum(h1_ref[...] * scale + shift, 0.0)
    o_ref[...] = lax.dot_general(h2, w2_ref[...], (((1,), (1,)), ((), ())),
                                 preferred_element_type=jnp.float32) + b2_ref[...]


def kernel(x, edge_index, W1, b1, gamma, beta, W2, b2, eps):
    row = edge_index[0]
    col = edge_index[1]
    pad = _EPAD - _E
    trash = _N + (jnp.arange(pad, dtype=jnp.int32) % (_NROWS - _N))
    e0 = 16 * _CA * _CH

    def _split(flat):
        p0 = flat[:e0].reshape(16, _CA, _CH)
        p1 = flat[e0:].reshape(16, _CB, _CH)
        if _CA < _CHUNKS:
            p0 = jnp.concatenate(
                [p0, jnp.zeros((16, _CHUNKS - _CA, _CH), jnp.int32)], axis=1)
        if _CB < _CHUNKS:
            p1 = jnp.concatenate(
                [p1, jnp.zeros((16, _CHUNKS - _CB, _CH), jnp.int32)], axis=1)
        return jnp.concatenate([p0, p1], axis=0)

    row_p = _split(jnp.concatenate([row, jnp.zeros((pad,), jnp.int32)]))
    col_p = _split(jnp.concatenate([col, trash]))

    rx = pl.pallas_call(
        _relu_body,
        grid=(_GRID,),
        in_specs=[pl.BlockSpec((_BLK, _D), lambda i: (i, 0))],
        out_specs=pl.BlockSpec((_BLK, _D), lambda i: (i, 0)),
        out_shape=jax.ShapeDtypeStruct((_N, _D), jnp.float32),
    )(x)

    agg2 = _sc_agg()(rx, row_p, col_p)

    h1, st = pl.pallas_call(
        _mlp1_body,
        grid=(_GRID,),
        in_specs=[
            pl.BlockSpec(memory_space=pltpu.SMEM),
            pl.BlockSpec((_BLK, _D), lambda i: (i, 0)),
            pl.BlockSpec((2, _BLK, _D), lambda i: (0, i, 0)),
            pl.BlockSpec((2 * _D, _D), lambda i: (0, 0)),
            pl.BlockSpec((1, 2 * _D), lambda i: (0, 0)),
        ],
        out_specs=[
            pl.BlockSpec((_BLK, 2 * _D), lambda i: (i, 0)),
            pl.BlockSpec((2, 2 * _D), lambda i: (0, 0)),
        ],
        out_shape=[
            jax.ShapeDtypeStruct((_N, 2 * _D), jnp.float32),
            jax.ShapeDtypeStruct((2, 2 * _D), jnp.float32),
        ],
    )(eps.reshape(1, 1), x, agg2, W1, b1.reshape(1, 2 * _D))

    out = pl.pallas_call(
        _mlp2_body,
        grid=(_GRID,),
        in_specs=[
            pl.BlockSpec((2, 2 * _D), lambda i: (0, 0)),
            pl.BlockSpec((1, 2 * _D), lambda i: (0, 0)),
            pl.BlockSpec((1, 2 * _D), lambda i: (0, 0)),
            pl.BlockSpec((_BLK, 2 * _D), lambda i: (i, 0)),
            pl.BlockSpec((_D, 2 * _D), lambda i: (0, 0)),
            pl.BlockSpec((1, _D), lambda i: (0, 0)),
        ],
        out_specs=pl.BlockSpec((_BLK, _D), lambda i: (i, 0)),
        out_shape=jax.ShapeDtypeStruct((_N, _D), jnp.float32),
    )(st, gamma.reshape(1, 2 * _D), beta.reshape(1, 2 * _D), h1, W2,
      b2.reshape(1, _D))

    return out
